# baseline (speedup 1.0000x reference)
import functools
import numpy as np
import jax
import jax.numpy as jnp
from jax import lax
from jax.experimental import pallas as pl
from jax.experimental.pallas import tpu as pltpu

H, DH, ROT = 16, 64, 32
NE, NEP = 7, 8
BASE = 10000.0


def _dot(a, b, dims):
    return lax.dot_general(a, b, (dims, ((), ())),
                           preferred_element_type=jnp.float32)


def _rot_perm():
    hrot = H * ROT
    i = np.arange(hrot)[:, None]
    j = np.arange(hrot)[None, :]
    same = (i // ROT) == (j // ROT)
    ci, cj = i % ROT, j % ROT
    p = np.where(same & (cj < ROT // 2) & (ci == cj + ROT // 2), -1.0, 0.0)
    p = p + np.where(same & (cj >= ROT // 2) & (ci == cj - ROT // 2), 1.0, 0.0)
    return jnp.asarray(p, jnp.float32)


def _proj_body(TB, T, x_ref, ln1_ref, wkv_ref, wq_ref, wku_ref, wqu_ref,
               wvu_ref, wrq_ref, wrk_ref, p_ref,
               qr_ref, qn_ref, kr_ref, kn_ref, v_ref):
    pid = pl.program_id(0)
    x = x_ref[...]
    h = x * lax.rsqrt(jnp.mean(x * x, axis=1, keepdims=True) + 1e-6) * ln1_ref[...]
    kvl = _dot(h, wkv_ref[...], ((1,), (1,)))
    ql = _dot(h, wq_ref[...], ((1,), (1,)))
    qn_ref[...] = _dot(ql, wqu_ref[...], ((1,), (1,)))
    kn_ref[...] = _dot(kvl, wku_ref[...], ((1,), (1,)))
    v_ref[...] = _dot(kvl, wvu_ref[...], ((1,), (1,)))
    qr = _dot(ql, wrq_ref[...], ((1,), (1,)))
    kr = _dot(h, wrk_ref[...], ((1,), (1,)))
    blocks_per_seq = T // TB
    t0 = (pid % blocks_per_seq) * TB
    trow = (t0 + lax.broadcasted_iota(jnp.int32, (TB, H * ROT), 0)).astype(jnp.float32)
    c = lax.broadcasted_iota(jnp.int32, (TB, H * ROT), 1)
    fidx = jnp.mod(c, ROT // 2).astype(jnp.float32)
    invf = jnp.exp(fidx * (-np.log(BASE) / (ROT // 2)))
    ang = trow * invf
    cos = jnp.cos(ang)
    sin = jnp.sin(ang)
    p = p_ref[...]
    qr_ref[...] = qr * cos + _dot(qr, p, ((1,), (0,))) * sin
    kr_ref[...] = kr * cos + _dot(kr, p, ((1,), (0,))) * sin


def _attn_body(BQ, T, qr_ref, qn_ref, kr_ref, kn_ref, v_ref, o_ref):
    qi = pl.program_id(2)
    s = _dot(qr_ref[0, 0], kr_ref[0, 0], ((1,), (1,)))
    s = s + _dot(qn_ref[0, 0], kn_ref[0, 0], ((1,), (1,)))
    s = s * (1.0 / np.sqrt(DH))
    row = qi * BQ + lax.broadcasted_iota(jnp.int32, (BQ, T), 0)
    col = lax.broadcasted_iota(jnp.int32, (BQ, T), 1)
    s = jnp.where(col <= row, s, -jnp.inf)
    m = jnp.max(s, axis=1, keepdims=True)
    p = jnp.exp(s - m)
    p = p / jnp.sum(p, axis=1, keepdims=True)
    o_ref[0, 0] = _dot(p, v_ref[0, 0], ((1,), (0,)))


def _post_body(x_ref, y_ref, wo_ref, ln2_ref, shg_ref, shu_ref, shd_ref,
               wr_ref, rb_ref, base_ref, h2_ref, wts_ref):
    x2 = x_ref[...] + _dot(y_ref[...], wo_ref[...], ((1,), (1,)))
    h2 = x2 * lax.rsqrt(jnp.mean(x2 * x2, axis=1, keepdims=True) + 1e-6) * ln2_ref[...]
    h2_ref[...] = h2
    sg = _dot(h2, shg_ref[...], ((1,), (1,)))
    su = _dot(h2, shu_ref[...], ((1,), (1,)))
    act = sg * jax.nn.sigmoid(sg) * su
    base_ref[...] = x2 + _dot(act, shd_ref[...], ((1,), (1,)))
    logits = _dot(h2, wr_ref[...], ((1,), (1,))) + rb_ref[...]
    probs = jax.nn.sigmoid(logits)
    colv = lax.broadcasted_iota(jnp.int32, probs.shape, 1).astype(jnp.float32)
    m1 = jnp.max(probs, axis=1, keepdims=True)
    i1 = jnp.min(jnp.where(probs == m1, colv, 99.0), axis=1, keepdims=True)
    p2 = jnp.where(colv == i1, -1.0, probs)
    m2 = jnp.max(p2, axis=1, keepdims=True)
    i2 = jnp.min(jnp.where(p2 == m2, colv, 99.0), axis=1, keepdims=True)
    ssum = m1 + m2
    wts_ref[...] = (jnp.where(colv == i1, m1 / ssum, 0.0)
                    + jnp.where(colv == i2, m2 / ssum, 0.0))


def _moe_body(base_ref, h2_ref, wts_ref, rg_ref, ru_ref, rd_ref, out_ref):
    e = pl.program_id(1)

    @pl.when(e == 0)
    def _():
        out_ref[...] = base_ref[...]

    h2 = h2_ref[...]
    g = _dot(h2, rg_ref[0], ((1,), (1,)))
    u = _dot(h2, ru_ref[0], ((1,), (1,)))
    act = g * jax.nn.sigmoid(g) * u
    eo = _dot(act, rd_ref[0], ((1,), (1,)))
    colv = lax.broadcasted_iota(jnp.int32, wts_ref.shape, 1).astype(jnp.float32)
    w = jnp.sum(wts_ref[...] * jnp.where(colv == e, 1.0, 0.0),
                axis=1, keepdims=True)
    out_ref[...] = out_ref[...] + w * eo


def kernel(x, ln1_w, ln2_w, w_kv_d, w_q_d, w_k_u, w_q_u, w_v_u, w_rope_q,
           w_rope_k, w_o, sh_gate, sh_up, sh_down, r_gate, r_up, r_down,
           w_router, routing_bias):
    b, t, d = x.shape
    n = b * t
    lat = w_kv_d.shape[0]
    i_dim = sh_gate.shape[0]
    xf = x.reshape(n, d)

    TB = min(512, t)
    grid1 = (n // TB,)
    fullspec = lambda shape: pl.BlockSpec(shape, lambda i: (0,) * len(shape))
    rowspec = lambda w: pl.BlockSpec((TB, w), lambda i: (i, 0))
    qr, qn, kr, kn, v = pl.pallas_call(
        functools.partial(_proj_body, TB, t),
        grid=grid1,
        in_specs=[
            rowspec(d), fullspec((1, d)), fullspec((lat, d)), fullspec((lat, d)),
            fullspec((H * DH, lat)), fullspec((H * DH, lat)), fullspec((H * DH, lat)),
            fullspec((H * ROT, lat)), fullspec((H * ROT, d)), fullspec((H * ROT, H * ROT)),
        ],
        out_specs=[rowspec(H * ROT), rowspec(H * DH), rowspec(H * ROT),
                   rowspec(H * DH), rowspec(H * DH)],
        out_shape=[
            jax.ShapeDtypeStruct((n, H * ROT), jnp.float32),
            jax.ShapeDtypeStruct((n, H * DH), jnp.float32),
            jax.ShapeDtypeStruct((n, H * ROT), jnp.float32),
            jax.ShapeDtypeStruct((n, H * DH), jnp.float32),
            jax.ShapeDtypeStruct((n, H * DH), jnp.float32),
        ],
    )(xf, ln1_w.reshape(1, d), w_kv_d, w_q_d, w_k_u, w_q_u, w_v_u,
      w_rope_q, w_rope_k, _rot_perm())

    BQ = min(256, t)
    qr4 = qr.reshape(b, t, H, ROT).transpose(0, 2, 1, 3)
    qn4 = qn.reshape(b, t, H, DH).transpose(0, 2, 1, 3)[..., ROT:]
    kr4 = kr.reshape(b, t, H, ROT).transpose(0, 2, 1, 3)
    kn4 = kn.reshape(b, t, H, DH).transpose(0, 2, 1, 3)[..., ROT:]
    v4 = v.reshape(b, t, H, DH).transpose(0, 2, 1, 3)
    y4 = pl.pallas_call(
        functools.partial(_attn_body, BQ, t),
        grid=(b, H, t // BQ),
        in_specs=[
            pl.BlockSpec((1, 1, BQ, ROT), lambda bb, hh, qi: (bb, hh, qi, 0)),
            pl.BlockSpec((1, 1, BQ, DH - ROT), lambda bb, hh, qi: (bb, hh, qi, 0)),
            pl.BlockSpec((1, 1, t, ROT), lambda bb, hh, qi: (bb, hh, 0, 0)),
            pl.BlockSpec((1, 1, t, DH - ROT), lambda bb, hh, qi: (bb, hh, 0, 0)),
            pl.BlockSpec((1, 1, t, DH), lambda bb, hh, qi: (bb, hh, 0, 0)),
        ],
        out_specs=pl.BlockSpec((1, 1, BQ, DH), lambda bb, hh, qi: (bb, hh, qi, 0)),
        out_shape=jax.ShapeDtypeStruct((b, H, t, DH), jnp.float32),
        compiler_params=pltpu.CompilerParams(
            dimension_semantics=("parallel", "parallel", "parallel")),
    )(qr4, qn4, kr4, kn4, v4)
    y = y4.transpose(0, 2, 1, 3)

    TB3 = min(512, t)
    wr_pad = jnp.zeros((NEP, d), jnp.float32).at[:NE].set(w_router)
    rb_pad = jnp.full((1, NEP), -1e30, jnp.float32).at[0, :NE].set(routing_bias)
    rowspec3 = lambda w: pl.BlockSpec((TB3, w), lambda i: (i, 0))
    base, h2, wts = pl.pallas_call(
        _post_body,
        grid=(n // TB3,),
        in_specs=[
            rowspec3(d), rowspec3(d), fullspec((d, d)), fullspec((1, d)),
            fullspec((i_dim, d)), fullspec((i_dim, d)), fullspec((d, i_dim)),
            fullspec((NEP, d)), fullspec((1, NEP)),
        ],
        out_specs=[rowspec3(d), rowspec3(d), rowspec3(NEP)],
        out_shape=[
            jax.ShapeDtypeStruct((n, d), jnp.float32),
            jax.ShapeDtypeStruct((n, d), jnp.float32),
            jax.ShapeDtypeStruct((n, NEP), jnp.float32),
        ],
    )(xf, y.reshape(n, d), w_o, ln2_w.reshape(1, d), sh_gate, sh_up, sh_down,
      wr_pad, rb_pad)

    TBM = min(1024, n)
    out = pl.pallas_call(
        _moe_body,
        grid=(n // TBM, NE),
        in_specs=[
            pl.BlockSpec((TBM, d), lambda i, e: (i, 0)),
            pl.BlockSpec((TBM, d), lambda i, e: (i, 0)),
            pl.BlockSpec((TBM, NEP), lambda i, e: (i, 0)),
            pl.BlockSpec((1, i_dim, d), lambda i, e: (e, 0, 0)),
            pl.BlockSpec((1, i_dim, d), lambda i, e: (e, 0, 0)),
            pl.BlockSpec((1, d, i_dim), lambda i, e: (e, 0, 0)),
        ],
        out_specs=pl.BlockSpec((TBM, d), lambda i, e: (i, 0)),
        out_shape=jax.ShapeDtypeStruct((n, d), jnp.float32),
        compiler_params=pltpu.CompilerParams(
            dimension_semantics=("parallel", "arbitrary")),
    )(base, h2, wts, r_gate, r_up, r_down)

    return out.reshape(b, t, d)
